# baseline (speedup 1.0000x reference)
import functools

import jax
import jax.numpy as jnp
from jax import lax
from jax.experimental import pallas as pl
from jax.experimental.pallas import tpu as pltpu
from jax.experimental.pallas import tpu_sc as plsc

K_DIM = 768
HEADS = 12
HD = K_DIM // HEADS
FF = 4 * K_DIM
SB = 512


def _sc_gather(table, idx_flat):
    n = idx_flat.shape[0]
    d = table.shape[1]
    info = plsc.get_sparse_core_info()
    nw = info.num_cores * info.num_subcores
    per_w = n // nw
    mesh = plsc.VectorSubcoreMesh(core_axis_name="c", subcore_axis_name="s")

    @functools.partial(
        pl.kernel,
        mesh=mesh,
        out_type=jax.ShapeDtypeStruct((n, d), table.dtype),
        scratch_types=[
            pltpu.VMEM((per_w,), jnp.int32),
            pltpu.VMEM((per_w, d), table.dtype),
            pltpu.SemaphoreType.DMA,
        ],
    )
    def gather_kernel(table_hbm, idx_hbm, out_hbm, idx_v, rows_v, sem):
        wid = lax.axis_index("s") * info.num_cores + lax.axis_index("c")
        base = wid * per_w
        pltpu.sync_copy(idx_hbm.at[pl.ds(base, per_w)], idx_v)
        pltpu.async_copy(table_hbm.at[idx_v], rows_v, sem).wait()
        pltpu.sync_copy(rows_v, out_hbm.at[pl.ds(base, per_w)])

    return gather_kernel(table, idx_flat)


def _add_pos_body(tok_ref, pos_ref, o_ref):
    o_ref[...] = tok_ref[...] + pos_ref[...]


def _attn_body(xq_ref, xkv_ref, wq_ref, wk_ref, wv_ref, o_ref, k_scr, v_scr):
    sb = pl.program_id(2)
    bf = jnp.bfloat16

    @pl.when(sb == 0)
    def _():
        xkv = xkv_ref[0].astype(bf)
        k_scr[...] = jnp.dot(xkv, wk_ref[0],
                             preferred_element_type=jnp.float32).astype(bf)
        v_scr[...] = jnp.dot(xkv, wv_ref[0],
                             preferred_element_type=jnp.float32).astype(bf)

    q = jnp.dot(xq_ref[0].astype(bf), wq_ref[0],
                preferred_element_type=jnp.float32).astype(bf)
    s = lax.dot_general(q, k_scr[...], (((1,), (1,)), ((), ())),
                        preferred_element_type=jnp.float32) * (HD ** -0.5)
    m = jnp.max(s, axis=-1, keepdims=True)
    e = jnp.exp(s - m)
    acc = jnp.dot(e.astype(bf), v_scr[...],
                  preferred_element_type=jnp.float32)
    o_ref[0, 0] = acc / jnp.sum(e, axis=-1, keepdims=True)


def _ln(y, g, b):
    m = jnp.mean(y, axis=-1, keepdims=True)
    c = y - m
    v = jnp.mean(c * c, axis=-1, keepdims=True)
    return c * jax.lax.rsqrt(v + 1e-5) * g + b


def _ffn_body(oc_ref, x_ref, wo_ref, g1_ref, be1_ref, w1_ref, b1_ref,
              w2_ref, b2_ref, g2_ref, be2_ref, out_ref):
    bf = jnp.bfloat16
    y = jnp.dot(oc_ref[...].astype(bf), wo_ref[...],
                preferred_element_type=jnp.float32) + x_ref[...]
    y = _ln(y, g1_ref[...], be1_ref[...])
    f = jnp.maximum(
        jnp.dot(y.astype(bf), w1_ref[...], preferred_element_type=jnp.float32)
        + b1_ref[...], 0.0)
    z = jnp.dot(f.astype(bf), w2_ref[...], preferred_element_type=jnp.float32) \
        + b2_ref[...] + y
    out_ref[...] = _ln(z, g2_ref[...], be2_ref[...])


def _head_body(x_ref, wc_ref, bc_ref, o_ref):
    m = jnp.mean(x_ref[...], axis=1)
    logits = jnp.dot(m, wc_ref[...],
                     preferred_element_type=jnp.float32) + bc_ref[...]
    lmax = jnp.max(logits, axis=-1, keepdims=True)
    e = jnp.exp(logits - lmax)
    o_ref[...] = (logits - lmax) - jnp.log(jnp.sum(e, axis=-1, keepdims=True))


def _add_pos(tokg, pos):
    B, S, K = tokg.shape
    return pl.pallas_call(
        _add_pos_body,
        grid=(B,),
        in_specs=[
            pl.BlockSpec((1, S, K), lambda b: (b, 0, 0)),
            pl.BlockSpec((S, K), lambda b: (0, 0)),
        ],
        out_specs=pl.BlockSpec((1, S, K), lambda b: (b, 0, 0)),
        out_shape=jax.ShapeDtypeStruct((B, S, K), jnp.float32),
    )(tokg, pos)


def _attention(h, wq, wk, wv):
    B, S, K = h.shape
    nsb = S // SB
    return pl.pallas_call(
        _attn_body,
        grid=(B, HEADS, nsb),
        in_specs=[
            pl.BlockSpec((1, SB, K), lambda b, hh, sb: (b, sb, 0)),
            pl.BlockSpec((1, S, K), lambda b, hh, sb: (b, 0, 0)),
            pl.BlockSpec((1, K, HD), lambda b, hh, sb: (hh, 0, 0)),
            pl.BlockSpec((1, K, HD), lambda b, hh, sb: (hh, 0, 0)),
            pl.BlockSpec((1, K, HD), lambda b, hh, sb: (hh, 0, 0)),
        ],
        out_specs=pl.BlockSpec((1, 1, SB, HD),
                               lambda b, hh, sb: (b, hh, sb, 0)),
        out_shape=jax.ShapeDtypeStruct((B, HEADS, S, HD), jnp.float32),
        scratch_shapes=[
            pltpu.VMEM((S, HD), jnp.bfloat16),
            pltpu.VMEM((S, HD), jnp.bfloat16),
        ],
    )(h, h, wq, wk, wv)


def _ffn(oc, x, wo, g1, be1, w1, b1, w2, b2, g2, be2):
    N, K = oc.shape
    rb = 512
    vec = lambda a: a.reshape(1, -1)
    return pl.pallas_call(
        _ffn_body,
        grid=(N // rb,),
        in_specs=[
            pl.BlockSpec((rb, K), lambda i: (i, 0)),
            pl.BlockSpec((rb, K), lambda i: (i, 0)),
            pl.BlockSpec((K, K), lambda i: (0, 0)),
            pl.BlockSpec((1, K), lambda i: (0, 0)),
            pl.BlockSpec((1, K), lambda i: (0, 0)),
            pl.BlockSpec((K, FF), lambda i: (0, 0)),
            pl.BlockSpec((1, FF), lambda i: (0, 0)),
            pl.BlockSpec((FF, K), lambda i: (0, 0)),
            pl.BlockSpec((1, K), lambda i: (0, 0)),
            pl.BlockSpec((1, K), lambda i: (0, 0)),
            pl.BlockSpec((1, K), lambda i: (0, 0)),
        ],
        out_specs=pl.BlockSpec((rb, K), lambda i: (i, 0)),
        out_shape=jax.ShapeDtypeStruct((N, K), jnp.float32),
    )(oc, x, wo, vec(g1), vec(be1), w1, vec(b1), w2, vec(b2),
      vec(g2), vec(be2))


def _head(h, wc, bc):
    B, S, K = h.shape
    C = wc.shape[1]
    return pl.pallas_call(
        _head_body,
        grid=(1,),
        in_specs=[
            pl.BlockSpec((B, S, K), lambda i: (0, 0, 0)),
            pl.BlockSpec((K, C), lambda i: (0, 0)),
            pl.BlockSpec((1, C), lambda i: (0, 0)),
        ],
        out_specs=pl.BlockSpec((B, C), lambda i: (0, 0)),
        out_shape=jax.ShapeDtypeStruct((B, C), jnp.float32),
    )(h, wc, bc.reshape(1, -1))


def kernel(x, params):
    B, S = x.shape

    idx = x.reshape(-1).astype(jnp.int32)
    rows = _sc_gather(params["tok"], idx)
    h = _add_pos(rows.reshape(B, S, K_DIM), params["pos"])

    for p in params["layers"]:
        per_head = lambda w: (w.reshape(K_DIM, HEADS, HD)
                              .transpose(1, 0, 2).astype(jnp.bfloat16))
        o = _attention(h, per_head(p["Wq"]), per_head(p["Wk"]),
                       per_head(p["Wv"]))
        oc = o.transpose(0, 2, 1, 3).reshape(B * S, K_DIM)
        hf = _ffn(oc, h.reshape(B * S, K_DIM), p["Wo"].astype(jnp.bfloat16),
                  p["ln1_g"], p["ln1_b"], p["W1"].astype(jnp.bfloat16),
                  p["b1"], p["W2"].astype(jnp.bfloat16), p["b2"],
                  p["ln2_g"], p["ln2_b"])
        h = hf.reshape(B, S, K_DIM)

    return _head(h, params["Wc"], params["bc"])


# speedup vs baseline: 2.0350x; 2.0350x over previous
import functools

import jax
import jax.numpy as jnp
from jax import lax
from jax.experimental import pallas as pl
from jax.experimental.pallas import tpu as pltpu
from jax.experimental.pallas import tpu_sc as plsc

K_DIM = 768
HEADS = 12
HD = K_DIM // HEADS
FF = 4 * K_DIM
SB = 512


def _sc_gather(table, idx_flat):
    n = idx_flat.shape[0]
    d = table.shape[1]
    info = plsc.get_sparse_core_info()
    nw = info.num_cores * info.num_subcores
    per_w = n // nw
    mesh = plsc.VectorSubcoreMesh(core_axis_name="c", subcore_axis_name="s")

    @functools.partial(
        pl.kernel,
        mesh=mesh,
        out_type=jax.ShapeDtypeStruct((n, d), table.dtype),
        scratch_types=[
            pltpu.VMEM((per_w,), jnp.int32),
            pltpu.VMEM((per_w, d), table.dtype),
            pltpu.SemaphoreType.DMA,
        ],
    )
    def gather_kernel(table_hbm, idx_hbm, out_hbm, idx_v, rows_v, sem):
        wid = lax.axis_index("s") * info.num_cores + lax.axis_index("c")
        base = wid * per_w
        pltpu.sync_copy(idx_hbm.at[pl.ds(base, per_w)], idx_v)
        pltpu.async_copy(table_hbm.at[idx_v], rows_v, sem).wait()
        pltpu.sync_copy(rows_v, out_hbm.at[pl.ds(base, per_w)])

    return gather_kernel(table, idx_flat)


def _add_pos_body(tok_ref, pos_ref, o_ref):
    o_ref[...] = tok_ref[...] + pos_ref[...]


def _qkv_body(x_ref, wq_ref, wk_ref, wv_ref, q_ref, k_ref, va_ref):
    bf = jnp.bfloat16
    xb = x_ref[...].astype(bf)
    q = jnp.dot(xb, wq_ref[...], preferred_element_type=jnp.float32)
    q_ref[...] = (q * (HD ** -0.5)).astype(bf)
    k_ref[...] = jnp.dot(xb, wk_ref[...],
                         preferred_element_type=jnp.float32).astype(bf)
    v = jnp.dot(xb, wv_ref[...], preferred_element_type=jnp.float32)
    rb = v.shape[0]
    pad = jnp.concatenate(
        [jnp.ones((rb, 1), bf), jnp.zeros((rb, 128 - HD - 1), bf)], axis=1)
    for h in range(HEADS):
        va_ref[:, h * 128:h * 128 + HD] = v[:, h * HD:(h + 1) * HD].astype(bf)
        va_ref[:, h * 128 + HD:(h + 1) * 128] = pad


def _attn_body(q_ref, k_ref, va_ref, o_ref):
    bf = jnp.bfloat16
    for h in range(HEADS):
        qh = q_ref[:, h * HD:(h + 1) * HD]
        kh = k_ref[:, h * HD:(h + 1) * HD]
        s = lax.dot_general(qh, kh, (((1,), (1,)), ((), ())),
                            preferred_element_type=jnp.float32)
        e = jnp.exp(s).astype(bf)
        acc = jnp.dot(e, va_ref[:, h * 128:(h + 1) * 128],
                      preferred_element_type=jnp.float32)
        o_ref[:, h * HD:(h + 1) * HD] = \
            (acc[:, :HD] / acc[:, HD:HD + 1]).astype(bf)


def _ln(y, g, b):
    m = jnp.mean(y, axis=-1, keepdims=True)
    c = y - m
    v = jnp.mean(c * c, axis=-1, keepdims=True)
    return c * jax.lax.rsqrt(v + 1e-5) * g + b


def _ffn_body(oc_ref, x_ref, wo_ref, g1_ref, be1_ref, w1_ref, b1_ref,
              w2_ref, b2_ref, g2_ref, be2_ref, out_ref):
    bf = jnp.bfloat16
    y = jnp.dot(oc_ref[...], wo_ref[...],
                preferred_element_type=jnp.float32) + x_ref[...]
    y = _ln(y, g1_ref[...], be1_ref[...])
    f = jnp.maximum(
        jnp.dot(y.astype(bf), w1_ref[...], preferred_element_type=jnp.float32)
        + b1_ref[...], 0.0)
    z = jnp.dot(f.astype(bf), w2_ref[...], preferred_element_type=jnp.float32) \
        + b2_ref[...] + y
    out_ref[...] = _ln(z, g2_ref[...], be2_ref[...])


def _head_body(x_ref, wc_ref, bc_ref, o_ref):
    m = jnp.mean(x_ref[...], axis=1)
    logits = jnp.dot(m, wc_ref[...],
                     preferred_element_type=jnp.float32) + bc_ref[...]
    lmax = jnp.max(logits, axis=-1, keepdims=True)
    e = jnp.exp(logits - lmax)
    o_ref[...] = (logits - lmax) - jnp.log(jnp.sum(e, axis=-1, keepdims=True))


def _add_pos(tokg, pos):
    B, S, K = tokg.shape
    return pl.pallas_call(
        _add_pos_body,
        grid=(B,),
        in_specs=[
            pl.BlockSpec((1, S, K), lambda b: (b, 0, 0)),
            pl.BlockSpec((S, K), lambda b: (0, 0)),
        ],
        out_specs=pl.BlockSpec((1, S, K), lambda b: (b, 0, 0)),
        out_shape=jax.ShapeDtypeStruct((B, S, K), jnp.float32),
    )(tokg, pos)


def _qkv(hflat, wq, wk, wv):
    N, K = hflat.shape
    rb = 512
    bf = jnp.bfloat16
    return pl.pallas_call(
        _qkv_body,
        grid=(N // rb,),
        in_specs=[
            pl.BlockSpec((rb, K), lambda i: (i, 0)),
            pl.BlockSpec((K, K), lambda i: (0, 0)),
            pl.BlockSpec((K, K), lambda i: (0, 0)),
            pl.BlockSpec((K, K), lambda i: (0, 0)),
        ],
        out_specs=[
            pl.BlockSpec((rb, K), lambda i: (i, 0)),
            pl.BlockSpec((rb, K), lambda i: (i, 0)),
            pl.BlockSpec((rb, HEADS * 128), lambda i: (i, 0)),
        ],
        out_shape=[
            jax.ShapeDtypeStruct((N, K), bf),
            jax.ShapeDtypeStruct((N, K), bf),
            jax.ShapeDtypeStruct((N, HEADS * 128), bf),
        ],
    )(hflat, wq, wk, wv)


def _attention(q, k, va, B, S):
    N, K = q.shape
    nsb = S // SB
    return pl.pallas_call(
        _attn_body,
        grid=(B, nsb),
        in_specs=[
            pl.BlockSpec((SB, K), lambda b, sb: (b * nsb + sb, 0)),
            pl.BlockSpec((S, K), lambda b, sb: (b, 0)),
            pl.BlockSpec((S, HEADS * 128), lambda b, sb: (b, 0)),
        ],
        out_specs=pl.BlockSpec((SB, K), lambda b, sb: (b * nsb + sb, 0)),
        out_shape=jax.ShapeDtypeStruct((N, K), jnp.bfloat16),
    )(q, k, va)


def _ffn(oc, x, wo, g1, be1, w1, b1, w2, b2, g2, be2):
    N, K = oc.shape
    rb = 512
    vec = lambda a: a.reshape(1, -1)
    return pl.pallas_call(
        _ffn_body,
        grid=(N // rb,),
        in_specs=[
            pl.BlockSpec((rb, K), lambda i: (i, 0)),
            pl.BlockSpec((rb, K), lambda i: (i, 0)),
            pl.BlockSpec((K, K), lambda i: (0, 0)),
            pl.BlockSpec((1, K), lambda i: (0, 0)),
            pl.BlockSpec((1, K), lambda i: (0, 0)),
            pl.BlockSpec((K, FF), lambda i: (0, 0)),
            pl.BlockSpec((1, FF), lambda i: (0, 0)),
            pl.BlockSpec((FF, K), lambda i: (0, 0)),
            pl.BlockSpec((1, K), lambda i: (0, 0)),
            pl.BlockSpec((1, K), lambda i: (0, 0)),
            pl.BlockSpec((1, K), lambda i: (0, 0)),
        ],
        out_specs=pl.BlockSpec((rb, K), lambda i: (i, 0)),
        out_shape=jax.ShapeDtypeStruct((N, K), jnp.float32),
    )(oc, x, wo, vec(g1), vec(be1), w1, vec(b1), w2, vec(b2),
      vec(g2), vec(be2))


def _head(h, wc, bc):
    B, S, K = h.shape
    C = wc.shape[1]
    return pl.pallas_call(
        _head_body,
        grid=(1,),
        in_specs=[
            pl.BlockSpec((B, S, K), lambda i: (0, 0, 0)),
            pl.BlockSpec((K, C), lambda i: (0, 0)),
            pl.BlockSpec((1, C), lambda i: (0, 0)),
        ],
        out_specs=pl.BlockSpec((B, C), lambda i: (0, 0)),
        out_shape=jax.ShapeDtypeStruct((B, C), jnp.float32),
    )(h, wc, bc.reshape(1, -1))


def kernel(x, params):
    B, S = x.shape

    idx = x.reshape(-1).astype(jnp.int32)
    rows = _sc_gather(params["tok"], idx)
    h = _add_pos(rows.reshape(B, S, K_DIM), params["pos"])

    hflat = h.reshape(B * S, K_DIM)
    for p in params["layers"]:
        bfc = lambda w: w.astype(jnp.bfloat16)
        q, k, va = _qkv(hflat, bfc(p["Wq"]), bfc(p["Wk"]), bfc(p["Wv"]))
        oc = _attention(q, k, va, B, S)
        hflat = _ffn(oc, hflat, bfc(p["Wo"]),
                     p["ln1_g"], p["ln1_b"], bfc(p["W1"]), p["b1"],
                     bfc(p["W2"]), p["b2"], p["ln2_g"], p["ln2_b"])
    h = hflat.reshape(B, S, K_DIM)

    return _head(h, params["Wc"], params["bc"])


# speedup vs baseline: 2.1258x; 1.0446x over previous
import functools

import jax
import jax.numpy as jnp
from jax import lax
from jax.experimental import pallas as pl
from jax.experimental.pallas import tpu as pltpu
from jax.experimental.pallas import tpu_sc as plsc

K_DIM = 768
HEADS = 12
HD = K_DIM // HEADS
FF = 4 * K_DIM
SB = 512


def _sc_gather(table, idx_flat):
    n = idx_flat.shape[0]
    d = table.shape[1]
    info = plsc.get_sparse_core_info()
    nw = info.num_cores * info.num_subcores
    per_w = n // nw
    mesh = plsc.VectorSubcoreMesh(core_axis_name="c", subcore_axis_name="s")

    @functools.partial(
        pl.kernel,
        mesh=mesh,
        out_type=jax.ShapeDtypeStruct((n, d), table.dtype),
        scratch_types=[
            pltpu.VMEM((per_w,), jnp.int32),
            pltpu.VMEM((per_w, d), table.dtype),
            pltpu.SemaphoreType.DMA,
        ],
    )
    def gather_kernel(table_hbm, idx_hbm, out_hbm, idx_v, rows_v, sem):
        wid = lax.axis_index("s") * info.num_cores + lax.axis_index("c")
        base = wid * per_w
        pltpu.sync_copy(idx_hbm.at[pl.ds(base, per_w)], idx_v)
        pltpu.async_copy(table_hbm.at[idx_v], rows_v, sem).wait()
        pltpu.sync_copy(rows_v, out_hbm.at[pl.ds(base, per_w)])

    return gather_kernel(table, idx_flat)


def _add_pos_body(tok_ref, pos_ref, o_ref):
    o_ref[...] = tok_ref[...] + pos_ref[...]


def _attn_body(x_ref, wq_ref, wk_ref, wv_ref, o_ref, q_scr, k_scr, va_scr):
    sb = pl.program_id(1)
    bf = jnp.bfloat16

    @pl.when(sb == 0)
    def _():
        xb = x_ref[...].astype(bf)
        q = jnp.dot(xb, wq_ref[...].astype(bf),
                    preferred_element_type=jnp.float32)
        q_scr[...] = (q * (HD ** -0.5)).astype(bf)
        k_scr[...] = jnp.dot(xb, wk_ref[...].astype(bf),
                             preferred_element_type=jnp.float32).astype(bf)
        v = jnp.dot(xb, wv_ref[...].astype(bf),
                    preferred_element_type=jnp.float32)
        n = v.shape[0]
        pad = jnp.concatenate(
            [jnp.ones((n, 1), bf), jnp.zeros((n, 128 - HD - 1), bf)], axis=1)
        for h in range(HEADS):
            va_scr[:, h * 128:h * 128 + HD] = \
                v[:, h * HD:(h + 1) * HD].astype(bf)
            va_scr[:, h * 128 + HD:(h + 1) * 128] = pad

    rows = pl.ds(sb * SB, SB)
    for h in range(HEADS):
        qh = q_scr[rows, h * HD:(h + 1) * HD]
        kh = k_scr[:, h * HD:(h + 1) * HD]
        s = lax.dot_general(qh, kh, (((1,), (1,)), ((), ())),
                            preferred_element_type=jnp.float32)
        e = jnp.exp(s).astype(bf)
        acc = jnp.dot(e, va_scr[:, h * 128:(h + 1) * 128],
                      preferred_element_type=jnp.float32)
        o_ref[:, h * HD:(h + 1) * HD] = \
            (acc[:, :HD] / acc[:, HD:HD + 1]).astype(bf)


def _ln(y, g, b):
    m = jnp.mean(y, axis=-1, keepdims=True)
    c = y - m
    v = jnp.mean(c * c, axis=-1, keepdims=True)
    return c * jax.lax.rsqrt(v + 1e-5) * g + b


def _ffn_body(oc_ref, x_ref, wo_ref, g1_ref, be1_ref, w1_ref, b1_ref,
              w2_ref, b2_ref, g2_ref, be2_ref, out_ref):
    bf = jnp.bfloat16
    y = jnp.dot(oc_ref[...], wo_ref[...],
                preferred_element_type=jnp.float32) + x_ref[...]
    y = _ln(y, g1_ref[...], be1_ref[...])
    f = jnp.maximum(
        jnp.dot(y.astype(bf), w1_ref[...], preferred_element_type=jnp.float32)
        + b1_ref[...], 0.0)
    z = jnp.dot(f.astype(bf), w2_ref[...], preferred_element_type=jnp.float32) \
        + b2_ref[...] + y
    out_ref[...] = _ln(z, g2_ref[...], be2_ref[...])


def _head_body(x_ref, wc_ref, bc_ref, o_ref):
    m = jnp.mean(x_ref[...], axis=1)
    logits = jnp.dot(m, wc_ref[...],
                     preferred_element_type=jnp.float32) + bc_ref[...]
    lmax = jnp.max(logits, axis=-1, keepdims=True)
    e = jnp.exp(logits - lmax)
    o_ref[...] = (logits - lmax) - jnp.log(jnp.sum(e, axis=-1, keepdims=True))


def _add_pos(tokg, pos):
    B, S, K = tokg.shape
    return pl.pallas_call(
        _add_pos_body,
        grid=(B,),
        in_specs=[
            pl.BlockSpec((1, S, K), lambda b: (b, 0, 0)),
            pl.BlockSpec((S, K), lambda b: (0, 0)),
        ],
        out_specs=pl.BlockSpec((1, S, K), lambda b: (b, 0, 0)),
        out_shape=jax.ShapeDtypeStruct((B, S, K), jnp.float32),
    )(tokg, pos)


def _attention(hflat, wq, wk, wv, B, S):
    N, K = hflat.shape
    nsb = S // SB
    return pl.pallas_call(
        _attn_body,
        grid=(B, nsb),
        in_specs=[
            pl.BlockSpec((S, K), lambda b, sb: (b, 0)),
            pl.BlockSpec((K, K), lambda b, sb: (0, 0)),
            pl.BlockSpec((K, K), lambda b, sb: (0, 0)),
            pl.BlockSpec((K, K), lambda b, sb: (0, 0)),
        ],
        out_specs=pl.BlockSpec((SB, K), lambda b, sb: (b * nsb + sb, 0)),
        out_shape=jax.ShapeDtypeStruct((N, K), jnp.bfloat16),
        scratch_shapes=[
            pltpu.VMEM((S, K), jnp.bfloat16),
            pltpu.VMEM((S, K), jnp.bfloat16),
            pltpu.VMEM((S, HEADS * 128), jnp.bfloat16),
        ],
    )(hflat, wq, wk, wv)


def _ffn(oc, x, wo, g1, be1, w1, b1, w2, b2, g2, be2):
    N, K = oc.shape
    rb = 512
    vec = lambda a: a.reshape(1, -1)
    return pl.pallas_call(
        _ffn_body,
        grid=(N // rb,),
        in_specs=[
            pl.BlockSpec((rb, K), lambda i: (i, 0)),
            pl.BlockSpec((rb, K), lambda i: (i, 0)),
            pl.BlockSpec((K, K), lambda i: (0, 0)),
            pl.BlockSpec((1, K), lambda i: (0, 0)),
            pl.BlockSpec((1, K), lambda i: (0, 0)),
            pl.BlockSpec((K, FF), lambda i: (0, 0)),
            pl.BlockSpec((1, FF), lambda i: (0, 0)),
            pl.BlockSpec((FF, K), lambda i: (0, 0)),
            pl.BlockSpec((1, K), lambda i: (0, 0)),
            pl.BlockSpec((1, K), lambda i: (0, 0)),
            pl.BlockSpec((1, K), lambda i: (0, 0)),
        ],
        out_specs=pl.BlockSpec((rb, K), lambda i: (i, 0)),
        out_shape=jax.ShapeDtypeStruct((N, K), jnp.float32),
    )(oc, x, wo, vec(g1), vec(be1), w1, vec(b1), w2, vec(b2),
      vec(g2), vec(be2))


def _head(h, wc, bc):
    B, S, K = h.shape
    C = wc.shape[1]
    return pl.pallas_call(
        _head_body,
        grid=(1,),
        in_specs=[
            pl.BlockSpec((B, S, K), lambda i: (0, 0, 0)),
            pl.BlockSpec((K, C), lambda i: (0, 0)),
            pl.BlockSpec((1, C), lambda i: (0, 0)),
        ],
        out_specs=pl.BlockSpec((B, C), lambda i: (0, 0)),
        out_shape=jax.ShapeDtypeStruct((B, C), jnp.float32),
    )(h, wc, bc.reshape(1, -1))


def kernel(x, params):
    B, S = x.shape

    idx = x.reshape(-1).astype(jnp.int32)
    rows = _sc_gather(params["tok"], idx)
    h = _add_pos(rows.reshape(B, S, K_DIM), params["pos"])

    hflat = h.reshape(B * S, K_DIM)
    for p in params["layers"]:
        bfc = lambda w: w.astype(jnp.bfloat16)
        oc = _attention(hflat, p["Wq"], p["Wk"], p["Wv"], B, S)
        hflat = _ffn(oc, hflat, bfc(p["Wo"]),
                     p["ln1_g"], p["ln1_b"], bfc(p["W1"]), p["b1"],
                     bfc(p["W2"]), p["b2"], p["ln2_g"], p["ln2_b"])
    h = hflat.reshape(B, S, K_DIM)

    return _head(h, params["Wc"], params["bc"])


# speedup vs baseline: 2.4098x; 1.1336x over previous
import functools

import jax
import jax.numpy as jnp
from jax import lax
from jax.experimental import pallas as pl
from jax.experimental.pallas import tpu as pltpu
from jax.experimental.pallas import tpu_sc as plsc

K_DIM = 768
HEADS = 12
HD = K_DIM // HEADS
FF = 4 * K_DIM
SB = 512


def _sc_gather(table, idx_flat):
    n = idx_flat.shape[0]
    d = table.shape[1]
    info = plsc.get_sparse_core_info()
    nw = info.num_cores * info.num_subcores
    per_w = n // nw
    mesh = plsc.VectorSubcoreMesh(core_axis_name="c", subcore_axis_name="s")

    @functools.partial(
        pl.kernel,
        mesh=mesh,
        out_type=jax.ShapeDtypeStruct((n, d), table.dtype),
        scratch_types=[
            pltpu.VMEM((per_w,), jnp.int32),
            pltpu.VMEM((per_w, d), table.dtype),
            pltpu.SemaphoreType.DMA,
        ],
    )
    def gather_kernel(table_hbm, idx_hbm, out_hbm, idx_v, rows_v, sem):
        wid = lax.axis_index("s") * info.num_cores + lax.axis_index("c")
        base = wid * per_w
        pltpu.sync_copy(idx_hbm.at[pl.ds(base, per_w)], idx_v)
        pltpu.async_copy(table_hbm.at[idx_v], rows_v, sem).wait()
        pltpu.sync_copy(rows_v, out_hbm.at[pl.ds(base, per_w)])

    return gather_kernel(table, idx_flat)


def _add_pos_body(tok_ref, pos_ref, o_ref):
    o_ref[...] = tok_ref[...] + pos_ref[...]


def _attn_body(x_ref, wq_ref, wk_ref, wv_ref, o_ref, q_scr, k_scr, va_scr):
    sb = pl.program_id(1)
    bf = jnp.bfloat16
    f8 = jnp.float8_e4m3fn

    @pl.when(sb == 0)
    def _():
        xb = x_ref[...].astype(bf)
        q = jnp.dot(xb, wq_ref[...].astype(bf),
                    preferred_element_type=jnp.float32)
        q_scr[...] = (q * (HD ** -0.5)).astype(bf)
        k_scr[...] = jnp.dot(xb, wk_ref[...].astype(bf),
                             preferred_element_type=jnp.float32).astype(bf)
        v = jnp.dot(xb, wv_ref[...].astype(bf),
                    preferred_element_type=jnp.float32)
        n = v.shape[0]
        pad = jnp.concatenate(
            [jnp.ones((n, 1), f8), jnp.zeros((n, 128 - HD - 1), f8)], axis=1)
        for h in range(HEADS):
            va_scr[:, h * 128:h * 128 + HD] = \
                v[:, h * HD:(h + 1) * HD].astype(f8)
            va_scr[:, h * 128 + HD:(h + 1) * 128] = pad

    rows = pl.ds(sb * SB, SB)
    for h in range(HEADS):
        qh = q_scr[rows, h * HD:(h + 1) * HD]
        kh = k_scr[:, h * HD:(h + 1) * HD]
        s = lax.dot_general(qh, kh, (((1,), (1,)), ((), ())),
                            preferred_element_type=jnp.float32)
        e = jnp.exp(s).astype(f8)
        acc = jnp.dot(e, va_scr[:, h * 128:(h + 1) * 128],
                      preferred_element_type=jnp.float32)
        o_ref[:, h * HD:(h + 1) * HD] = \
            (acc[:, :HD] / acc[:, HD:HD + 1]).astype(bf)


def _ln(y, g, b):
    m = jnp.mean(y, axis=-1, keepdims=True)
    c = y - m
    v = jnp.mean(c * c, axis=-1, keepdims=True)
    return c * jax.lax.rsqrt(v + 1e-5) * g + b


def _ffn_body(oc_ref, x_ref, wo_ref, g1_ref, be1_ref, w1_ref, b1_ref,
              w2_ref, b2_ref, g2_ref, be2_ref, out_ref):
    bf = jnp.bfloat16
    nh = oc_ref.shape[0] // 2
    for i in range(2):
        r = slice(i * nh, (i + 1) * nh)
        y = jnp.dot(oc_ref[r, :], wo_ref[...],
                    preferred_element_type=jnp.float32) + x_ref[r, :]
        y = _ln(y, g1_ref[...], be1_ref[...])
        f = jnp.maximum(
            jnp.dot(y.astype(bf), w1_ref[...],
                    preferred_element_type=jnp.float32) + b1_ref[...], 0.0)
        z = jnp.dot(f.astype(bf), w2_ref[...],
                    preferred_element_type=jnp.float32) + b2_ref[...] + y
        out_ref[r, :] = _ln(z, g2_ref[...], be2_ref[...])


def _head_body(x_ref, wc_ref, bc_ref, o_ref):
    m = jnp.mean(x_ref[...], axis=1)
    logits = jnp.dot(m, wc_ref[...],
                     preferred_element_type=jnp.float32) + bc_ref[...]
    lmax = jnp.max(logits, axis=-1, keepdims=True)
    e = jnp.exp(logits - lmax)
    o_ref[...] = (logits - lmax) - jnp.log(jnp.sum(e, axis=-1, keepdims=True))


def _add_pos(tokg, pos):
    B, S, K = tokg.shape
    return pl.pallas_call(
        _add_pos_body,
        grid=(B,),
        in_specs=[
            pl.BlockSpec((1, S, K), lambda b: (b, 0, 0)),
            pl.BlockSpec((S, K), lambda b: (0, 0)),
        ],
        out_specs=pl.BlockSpec((1, S, K), lambda b: (b, 0, 0)),
        out_shape=jax.ShapeDtypeStruct((B, S, K), jnp.float32),
    )(tokg, pos)


def _attention(hflat, wq, wk, wv, B, S):
    N, K = hflat.shape
    nsb = S // SB
    return pl.pallas_call(
        _attn_body,
        grid=(B, nsb),
        in_specs=[
            pl.BlockSpec((S, K), lambda b, sb: (b, 0)),
            pl.BlockSpec((K, K), lambda b, sb: (0, 0)),
            pl.BlockSpec((K, K), lambda b, sb: (0, 0)),
            pl.BlockSpec((K, K), lambda b, sb: (0, 0)),
        ],
        out_specs=pl.BlockSpec((SB, K), lambda b, sb: (b * nsb + sb, 0)),
        out_shape=jax.ShapeDtypeStruct((N, K), jnp.bfloat16),
        scratch_shapes=[
            pltpu.VMEM((S, K), jnp.bfloat16),
            pltpu.VMEM((S, K), jnp.bfloat16),
            pltpu.VMEM((S, HEADS * 128), jnp.float8_e4m3fn),
        ],
    )(hflat, wq, wk, wv)


def _ffn(oc, x, wo, g1, be1, w1, b1, w2, b2, g2, be2):
    N, K = oc.shape
    rb = 512
    vec = lambda a: a.reshape(1, -1)
    return pl.pallas_call(
        _ffn_body,
        grid=(N // rb,),
        in_specs=[
            pl.BlockSpec((rb, K), lambda i: (i, 0)),
            pl.BlockSpec((rb, K), lambda i: (i, 0)),
            pl.BlockSpec((K, K), lambda i: (0, 0)),
            pl.BlockSpec((1, K), lambda i: (0, 0)),
            pl.BlockSpec((1, K), lambda i: (0, 0)),
            pl.BlockSpec((K, FF), lambda i: (0, 0)),
            pl.BlockSpec((1, FF), lambda i: (0, 0)),
            pl.BlockSpec((FF, K), lambda i: (0, 0)),
            pl.BlockSpec((1, K), lambda i: (0, 0)),
            pl.BlockSpec((1, K), lambda i: (0, 0)),
            pl.BlockSpec((1, K), lambda i: (0, 0)),
        ],
        out_specs=pl.BlockSpec((rb, K), lambda i: (i, 0)),
        out_shape=jax.ShapeDtypeStruct((N, K), jnp.float32),
    )(oc, x, wo, vec(g1), vec(be1), w1, vec(b1), w2, vec(b2),
      vec(g2), vec(be2))


def _head(h, wc, bc):
    B, S, K = h.shape
    C = wc.shape[1]
    return pl.pallas_call(
        _head_body,
        grid=(1,),
        in_specs=[
            pl.BlockSpec((B, S, K), lambda i: (0, 0, 0)),
            pl.BlockSpec((K, C), lambda i: (0, 0)),
            pl.BlockSpec((1, C), lambda i: (0, 0)),
        ],
        out_specs=pl.BlockSpec((B, C), lambda i: (0, 0)),
        out_shape=jax.ShapeDtypeStruct((B, C), jnp.float32),
    )(h, wc, bc.reshape(1, -1))


def kernel(x, params):
    B, S = x.shape

    idx = x.reshape(-1).astype(jnp.int32)
    rows = _sc_gather(params["tok"], idx)
    h = _add_pos(rows.reshape(B, S, K_DIM), params["pos"])

    hflat = h.reshape(B * S, K_DIM)
    for p in params["layers"]:
        bfc = lambda w: w.astype(jnp.bfloat16)
        oc = _attention(hflat, p["Wq"], p["Wk"], p["Wv"], B, S)
        hflat = _ffn(oc, hflat, bfc(p["Wo"]),
                     p["ln1_g"], p["ln1_b"], bfc(p["W1"]), p["b1"],
                     bfc(p["W2"]), p["b2"], p["ln2_g"], p["ln2_b"])
    h = hflat.reshape(B, S, K_DIM)

    return _head(h, params["Wc"], params["bc"])


# speedup vs baseline: 2.6214x; 1.0878x over previous
import functools

import jax
import jax.numpy as jnp
from jax import lax
from jax.experimental import pallas as pl
from jax.experimental.pallas import tpu as pltpu
from jax.experimental.pallas import tpu_sc as plsc

K_DIM = 768
HEADS = 12
HD = K_DIM // HEADS
FF = 4 * K_DIM
SB = 512


def _sc_gather(table, idx_flat):
    n = idx_flat.shape[0]
    d = table.shape[1]
    info = plsc.get_sparse_core_info()
    nw = info.num_cores * info.num_subcores
    per_w = n // nw
    mesh = plsc.VectorSubcoreMesh(core_axis_name="c", subcore_axis_name="s")

    @functools.partial(
        pl.kernel,
        mesh=mesh,
        out_type=jax.ShapeDtypeStruct((n, d), table.dtype),
        scratch_types=[
            pltpu.VMEM((per_w,), jnp.int32),
            pltpu.VMEM((per_w, d), table.dtype),
            pltpu.SemaphoreType.DMA,
        ],
    )
    def gather_kernel(table_hbm, idx_hbm, out_hbm, idx_v, rows_v, sem):
        wid = lax.axis_index("s") * info.num_cores + lax.axis_index("c")
        base = wid * per_w
        pltpu.sync_copy(idx_hbm.at[pl.ds(base, per_w)], idx_v)
        pltpu.async_copy(table_hbm.at[idx_v], rows_v, sem).wait()
        pltpu.sync_copy(rows_v, out_hbm.at[pl.ds(base, per_w)])

    return gather_kernel(table, idx_flat)


def _add_pos_body(tok_ref, pos_ref, o_ref):
    o_ref[...] = tok_ref[...] + pos_ref[...]


def _attn_body(x_ref, wq_ref, wk_ref, wv_ref, o_ref, q_scr, k_scr, va_scr):
    sb = pl.program_id(1)
    bf = jnp.bfloat16
    f8 = jnp.float8_e4m3fn

    @pl.when(sb == 0)
    def _():
        xb = x_ref[...].astype(bf)
        q = jnp.dot(xb, wq_ref[...].astype(bf),
                    preferred_element_type=jnp.float32)
        q_scr[...] = (q * 16.0).astype(f8)
        k_scr[...] = (jnp.dot(xb, wk_ref[...].astype(bf),
                              preferred_element_type=jnp.float32)
                      * 16.0).astype(f8)
        v = jnp.dot(xb, wv_ref[...].astype(bf),
                    preferred_element_type=jnp.float32)
        n = v.shape[0]
        pad = jnp.concatenate(
            [jnp.ones((n, 1), f8), jnp.zeros((n, 128 - HD - 1), f8)], axis=1)
        for h in range(HEADS):
            va_scr[:, h * 128:h * 128 + HD] = \
                v[:, h * HD:(h + 1) * HD].astype(f8)
            va_scr[:, h * 128 + HD:(h + 1) * 128] = pad

    rows = pl.ds(sb * SB, SB)
    for h in range(HEADS):
        qh = q_scr[rows, h * HD:(h + 1) * HD]
        kh = k_scr[:, h * HD:(h + 1) * HD]
        s = lax.dot_general(qh, kh, (((1,), (1,)), ((), ())),
                            preferred_element_type=jnp.float32)
        e = jnp.exp(s * (1.0 / (256.0 * HD ** 0.5))).astype(f8)
        acc = jnp.dot(e, va_scr[:, h * 128:(h + 1) * 128],
                      preferred_element_type=jnp.float32)
        o_ref[:, h * HD:(h + 1) * HD] = \
            (acc[:, :HD] / acc[:, HD:HD + 1]).astype(bf)


def _ln(y, g, b):
    m = jnp.mean(y, axis=-1, keepdims=True)
    c = y - m
    v = jnp.mean(c * c, axis=-1, keepdims=True)
    return c * jax.lax.rsqrt(v + 1e-5) * g + b


def _ffn_body(oc_ref, x_ref, wo_ref, g1_ref, be1_ref, w1_ref, b1_ref,
              w2_ref, b2_ref, g2_ref, be2_ref, out_ref):
    bf = jnp.bfloat16
    nh = oc_ref.shape[0] // 2
    for i in range(2):
        r = slice(i * nh, (i + 1) * nh)
        y = jnp.dot(oc_ref[r, :], wo_ref[...],
                    preferred_element_type=jnp.float32) + x_ref[r, :]
        y = _ln(y, g1_ref[...], be1_ref[...])
        f = jnp.maximum(
            jnp.dot(y.astype(bf), w1_ref[...],
                    preferred_element_type=jnp.float32) + b1_ref[...], 0.0)
        z = jnp.dot(f.astype(bf), w2_ref[...],
                    preferred_element_type=jnp.float32) + b2_ref[...] + y
        out_ref[r, :] = _ln(z, g2_ref[...], be2_ref[...])


def _head_body(x_ref, wc_ref, bc_ref, o_ref):
    m = jnp.mean(x_ref[...], axis=1)
    logits = jnp.dot(m, wc_ref[...],
                     preferred_element_type=jnp.float32) + bc_ref[...]
    lmax = jnp.max(logits, axis=-1, keepdims=True)
    e = jnp.exp(logits - lmax)
    o_ref[...] = (logits - lmax) - jnp.log(jnp.sum(e, axis=-1, keepdims=True))


def _add_pos(tokg, pos):
    B, S, K = tokg.shape
    return pl.pallas_call(
        _add_pos_body,
        grid=(B,),
        in_specs=[
            pl.BlockSpec((1, S, K), lambda b: (b, 0, 0)),
            pl.BlockSpec((S, K), lambda b: (0, 0)),
        ],
        out_specs=pl.BlockSpec((1, S, K), lambda b: (b, 0, 0)),
        out_shape=jax.ShapeDtypeStruct((B, S, K), jnp.float32),
    )(tokg, pos)


def _attention(hflat, wq, wk, wv, B, S):
    N, K = hflat.shape
    nsb = S // SB
    return pl.pallas_call(
        _attn_body,
        grid=(B, nsb),
        in_specs=[
            pl.BlockSpec((S, K), lambda b, sb: (b, 0)),
            pl.BlockSpec((K, K), lambda b, sb: (0, 0)),
            pl.BlockSpec((K, K), lambda b, sb: (0, 0)),
            pl.BlockSpec((K, K), lambda b, sb: (0, 0)),
        ],
        out_specs=pl.BlockSpec((SB, K), lambda b, sb: (b * nsb + sb, 0)),
        out_shape=jax.ShapeDtypeStruct((N, K), jnp.bfloat16),
        scratch_shapes=[
            pltpu.VMEM((S, K), jnp.float8_e4m3fn),
            pltpu.VMEM((S, K), jnp.float8_e4m3fn),
            pltpu.VMEM((S, HEADS * 128), jnp.float8_e4m3fn),
        ],
    )(hflat, wq, wk, wv)


def _ffn(oc, x, wo, g1, be1, w1, b1, w2, b2, g2, be2):
    N, K = oc.shape
    rb = 512
    vec = lambda a: a.reshape(1, -1)
    return pl.pallas_call(
        _ffn_body,
        grid=(N // rb,),
        in_specs=[
            pl.BlockSpec((rb, K), lambda i: (i, 0)),
            pl.BlockSpec((rb, K), lambda i: (i, 0)),
            pl.BlockSpec((K, K), lambda i: (0, 0)),
            pl.BlockSpec((1, K), lambda i: (0, 0)),
            pl.BlockSpec((1, K), lambda i: (0, 0)),
            pl.BlockSpec((K, FF), lambda i: (0, 0)),
            pl.BlockSpec((1, FF), lambda i: (0, 0)),
            pl.BlockSpec((FF, K), lambda i: (0, 0)),
            pl.BlockSpec((1, K), lambda i: (0, 0)),
            pl.BlockSpec((1, K), lambda i: (0, 0)),
            pl.BlockSpec((1, K), lambda i: (0, 0)),
        ],
        out_specs=pl.BlockSpec((rb, K), lambda i: (i, 0)),
        out_shape=jax.ShapeDtypeStruct((N, K), jnp.float32),
    )(oc, x, wo, vec(g1), vec(be1), w1, vec(b1), w2, vec(b2),
      vec(g2), vec(be2))


def _head(h, wc, bc):
    B, S, K = h.shape
    C = wc.shape[1]
    return pl.pallas_call(
        _head_body,
        grid=(1,),
        in_specs=[
            pl.BlockSpec((B, S, K), lambda i: (0, 0, 0)),
            pl.BlockSpec((K, C), lambda i: (0, 0)),
            pl.BlockSpec((1, C), lambda i: (0, 0)),
        ],
        out_specs=pl.BlockSpec((B, C), lambda i: (0, 0)),
        out_shape=jax.ShapeDtypeStruct((B, C), jnp.float32),
    )(h, wc, bc.reshape(1, -1))


def kernel(x, params):
    B, S = x.shape

    idx = x.reshape(-1).astype(jnp.int32)
    rows = _sc_gather(params["tok"], idx)
    h = _add_pos(rows.reshape(B, S, K_DIM), params["pos"])

    hflat = h.reshape(B * S, K_DIM)
    for p in params["layers"]:
        bfc = lambda w: w.astype(jnp.bfloat16)
        oc = _attention(hflat, p["Wq"], p["Wk"], p["Wv"], B, S)
        hflat = _ffn(oc, hflat, bfc(p["Wo"]),
                     p["ln1_g"], p["ln1_b"], bfc(p["W1"]), p["b1"],
                     bfc(p["W2"]), p["b2"], p["ln2_g"], p["ln2_b"])
    h = hflat.reshape(B, S, K_DIM)

    return _head(h, params["Wc"], params["bc"])
